# baseline (speedup 1.0000x reference)
import functools

import jax
import jax.numpy as jnp
from jax import lax
from jax.experimental import pallas as pl
from jax.experimental.pallas import tpu as pltpu
from jax.experimental.pallas import tpu_sc as plsc

N = 10000
E = 320000
D = 128
R = 8
NB = 4

NC = 2
NS = 16
NW = NC * NS

K = 128
EP = 327680
EPW = EP // NW
PH = 2
CPP = EPW // (PH * K)
ZB = 640
ZL = N - (NS - 1) * ZB
NDUMMY = N + 640


BN = 2000


def _prep_body(x_ref, wc_ref, w_ref, lw_ref, bias_ref, y2_ref, hloop_ref):
    xb = x_ref[...]
    w = w_ref[...]
    cols = []
    for r in range(R):
        wr = wc_ref[r, 0] * w[0]
        for b in range(1, NB):
            wr = wr + wc_ref[r, b] * w[b]
        cols.append(wr)
    cols.append(lw_ref[...])
    wide = jnp.concatenate(cols, axis=1)
    out = jnp.dot(
        xb.astype(jnp.bfloat16),
        wide.astype(jnp.bfloat16),
        preferred_element_type=jnp.float32,
    )
    for r in range(R):
        y2_ref[:, r, :] = out[:, r * D:(r + 1) * D]
    hloop_ref[...] = out[:, R * D:] + bias_ref[...]


def _prep(x, w_comp, weight, loop_weight, h_bias2d):
    grid = N // BN
    return pl.pallas_call(
        _prep_body,
        grid=(grid,),
        in_specs=[
            pl.BlockSpec((BN, D), lambda i: (i, 0)),
            pl.BlockSpec(memory_space=pltpu.SMEM),
            pl.BlockSpec((NB, D, D), lambda i: (0, 0, 0)),
            pl.BlockSpec((D, D), lambda i: (0, 0)),
            pl.BlockSpec((1, D), lambda i: (0, 0)),
        ],
        out_specs=[
            pl.BlockSpec((BN, R, D), lambda i: (i, 0, 0)),
            pl.BlockSpec((BN, D), lambda i: (i, 0)),
        ],
        out_shape=[
            jax.ShapeDtypeStruct((N, R, D), jnp.float32),
            jax.ShapeDtypeStruct((N, D), jnp.float32),
        ],
    )(x, w_comp, weight, loop_weight, h_bias2d)


def _gidx_body(ei_ref, et_ref, gidx_ref, dstp_ref):
    pad = (EP - E) // D
    flat = lax.broadcasted_iota(jnp.int32, (pad, D), 0) * D + lax.broadcasted_iota(
        jnp.int32, (pad, D), 1
    )
    g = ei_ref[0] * R + et_ref[...]
    gidx_ref[...] = jnp.concatenate([g, flat % (N * R)], axis=0).reshape(
        NW, PH, CPP, K
    )
    dstp_ref[...] = jnp.concatenate([ei_ref[1], N + flat % 640], axis=0).reshape(
        NW, PH, CPP, K
    )


def _gidx(ei3d, et2d):
    return pl.pallas_call(
        _gidx_body,
        out_shape=[
            jax.ShapeDtypeStruct((NW, PH, CPP, K), jnp.int32),
            jax.ShapeDtypeStruct((NW, PH, CPP, K), jnp.int32),
        ],
    )(ei3d, et2d)


def _sc_body(y2_hbm, gidx_hbm, dst_hbm, zeros_hbm, out_hbm,
             gidx_v, dst_v, rows0, rows1, h_sh, sem0, sem1):
    cid = lax.axis_index("c")
    sid = lax.axis_index("s")
    wid = sid * NC + cid
    rbase = sid * ZB

    pltpu.async_copy(gidx_hbm.at[wid, 0], gidx_v, sem0)
    pltpu.async_copy(dst_hbm.at[wid, 0], dst_v, sem1)

    @pl.when(sid < NS - 1)
    def _():
        pltpu.sync_copy(zeros_hbm, h_sh.at[pl.ds(rbase, ZB)])

    @pl.when(sid == NS - 1)
    def _():
        pltpu.sync_copy(zeros_hbm.at[pl.ds(0, ZL)], h_sh.at[pl.ds(rbase, ZL)])

    plsc.subcore_barrier()

    bufs = ((rows0, sem0), (rows1, sem1))
    for h in range(PH):
        if h == 0:
            pltpu.make_async_copy(gidx_hbm.at[wid, 0], gidx_v, sem0).wait()
            pltpu.make_async_copy(dst_hbm.at[wid, 0], dst_v, sem1).wait()
        else:
            pltpu.sync_copy(gidx_hbm.at[wid, h], gidx_v)
            pltpu.sync_copy(dst_hbm.at[wid, h], dst_v)
        for b in range(2):
            pltpu.async_copy(y2_hbm.at[gidx_v.at[b]], bufs[b][0], bufs[b][1])

        def pair(i, carry):
            for b in range(2):
                j = 2 * i + b
                buf, sem = bufs[b]
                pltpu.make_async_copy(y2_hbm.at[gidx_v.at[j]], buf, sem).wait()
                pltpu.sync_copy(buf, h_sh.at[dst_v.at[j]], add=True)

                @pl.when(j + 2 < CPP)
                def _():
                    pltpu.async_copy(y2_hbm.at[gidx_v.at[j + 2]], buf, sem)

            return carry

        lax.fori_loop(0, CPP // 2, pair, 0)
    plsc.subcore_barrier()

    @pl.when(sid < NS - 1)
    def _():
        pltpu.sync_copy(h_sh.at[pl.ds(rbase, ZB)], out_hbm.at[cid].at[pl.ds(rbase, ZB)])

    @pl.when(sid == NS - 1)
    def _():
        pltpu.sync_copy(h_sh.at[pl.ds(rbase, ZL)], out_hbm.at[cid].at[pl.ds(rbase, ZL)])


@functools.cache
def _sc_scatter_kernel():
    return pl.kernel(
        _sc_body,
        out_type=jax.ShapeDtypeStruct((NC, N, D), jnp.float32),
        mesh=plsc.VectorSubcoreMesh(
            core_axis_name="c", subcore_axis_name="s", num_cores=NC, num_subcores=NS
        ),
        scratch_types=[
            pltpu.VMEM((CPP, K), jnp.int32),
            pltpu.VMEM((CPP, K), jnp.int32),
            pltpu.VMEM((K, D), jnp.float32),
            pltpu.VMEM((K, D), jnp.float32),
            pltpu.VMEM_SHARED((NDUMMY, D), jnp.float32),
            pltpu.SemaphoreType.DMA,
            pltpu.SemaphoreType.DMA,
        ],
    )


FBN = 2000


def _final_body(p_ref, hl_ref, out_ref):
    out_ref[...] = p_ref[0] + p_ref[1] + hl_ref[...]


def _final(partial, h_loop):
    return pl.pallas_call(
        _final_body,
        grid=(N // FBN,),
        in_specs=[
            pl.BlockSpec((NC, FBN, D), lambda i: (0, i, 0)),
            pl.BlockSpec((FBN, D), lambda i: (i, 0)),
        ],
        out_specs=pl.BlockSpec((FBN, D), lambda i: (i, 0)),
        out_shape=jax.ShapeDtypeStruct((N, D), jnp.float32),
    )(partial, h_loop)


def kernel(x, edge_index, etypes, weight, w_comp, loop_weight, h_bias):
    y2, h_loop = _prep(x, w_comp, weight, loop_weight, h_bias.reshape(1, D))
    gidx, dstp = _gidx(
        edge_index.reshape(2, E // D, D),
        etypes.reshape(E // D, D),
    )
    partial = _sc_scatter_kernel()(
        y2.reshape(N * R, D),
        gidx,
        dstp,
        jnp.zeros((ZB, D), jnp.float32),
    )
    return _final(partial, h_loop)


# speedup vs baseline: 1.0044x; 1.0044x over previous
import functools

import jax
import jax.numpy as jnp
from jax import lax
from jax.experimental import pallas as pl
from jax.experimental.pallas import tpu as pltpu
from jax.experimental.pallas import tpu_sc as plsc

N = 10000
E = 320000
D = 128
R = 8
NB = 4

NC = 2
NS = 16
NW = NC * NS

K = 128
EP = 327680
EPW = EP // NW
PH = 2
CPP = EPW // (PH * K)
ZB = 640
ZL = N - (NS - 1) * ZB
NDUMMY = N + 640


BN = 1000


def _prep_body(x_ref, wc_ref, w_ref, lw_ref, bias_ref, y2_ref, hloop_ref):
    xb = x_ref[...]
    w = w_ref[...]
    cols = []
    for r in range(R):
        wr = wc_ref[r, 0] * w[0]
        for b in range(1, NB):
            wr = wr + wc_ref[r, b] * w[b]
        cols.append(wr)
    cols.append(lw_ref[...])
    wide = jnp.concatenate(cols, axis=1)
    out = jnp.dot(
        xb.astype(jnp.bfloat16),
        wide.astype(jnp.bfloat16),
        preferred_element_type=jnp.float32,
    )
    for r in range(R):
        y2_ref[:, r, :] = out[:, r * D:(r + 1) * D]
    hloop_ref[...] = out[:, R * D:] + bias_ref[...]


def _prep(x, w_comp, weight, loop_weight, h_bias2d):
    grid = N // BN
    return pl.pallas_call(
        _prep_body,
        grid=(grid,),
        in_specs=[
            pl.BlockSpec((BN, D), lambda i: (i, 0)),
            pl.BlockSpec(memory_space=pltpu.SMEM),
            pl.BlockSpec((NB, D, D), lambda i: (0, 0, 0)),
            pl.BlockSpec((D, D), lambda i: (0, 0)),
            pl.BlockSpec((1, D), lambda i: (0, 0)),
        ],
        out_specs=[
            pl.BlockSpec((BN, R, D), lambda i: (i, 0, 0)),
            pl.BlockSpec((BN, D), lambda i: (i, 0)),
        ],
        out_shape=[
            jax.ShapeDtypeStruct((N, R, D), jnp.float32),
            jax.ShapeDtypeStruct((N, D), jnp.float32),
        ],
    )(x, w_comp, weight, loop_weight, h_bias2d)


def _gidx_body(ei_ref, et_ref, gidx_ref, dstp_ref):
    pad = (EP - E) // D
    flat = lax.broadcasted_iota(jnp.int32, (pad, D), 0) * D + lax.broadcasted_iota(
        jnp.int32, (pad, D), 1
    )
    g = ei_ref[0] * R + et_ref[...]
    gidx_ref[...] = jnp.concatenate([g, flat % (N * R)], axis=0).reshape(
        NW, PH, CPP, K
    )
    dstp_ref[...] = jnp.concatenate([ei_ref[1], N + flat % 640], axis=0).reshape(
        NW, PH, CPP, K
    )


def _gidx(ei3d, et2d):
    return pl.pallas_call(
        _gidx_body,
        out_shape=[
            jax.ShapeDtypeStruct((NW, PH, CPP, K), jnp.int32),
            jax.ShapeDtypeStruct((NW, PH, CPP, K), jnp.int32),
        ],
    )(ei3d, et2d)


def _sc_body(y2_hbm, gidx_hbm, dst_hbm, zeros_hbm, out_hbm,
             gidx_v, dst_v, rows0, rows1, h_sh, sem0, sem1):
    cid = lax.axis_index("c")
    sid = lax.axis_index("s")
    wid = sid * NC + cid
    rbase = sid * ZB

    pltpu.async_copy(gidx_hbm.at[wid, 0], gidx_v, sem0)
    pltpu.async_copy(dst_hbm.at[wid, 0], dst_v, sem1)

    @pl.when(sid < NS - 1)
    def _():
        pltpu.sync_copy(zeros_hbm, h_sh.at[pl.ds(rbase, ZB)])

    @pl.when(sid == NS - 1)
    def _():
        pltpu.sync_copy(zeros_hbm.at[pl.ds(0, ZL)], h_sh.at[pl.ds(rbase, ZL)])

    plsc.subcore_barrier()

    bufs = ((rows0, sem0), (rows1, sem1))
    for h in range(PH):
        if h == 0:
            pltpu.make_async_copy(gidx_hbm.at[wid, 0], gidx_v, sem0).wait()
            pltpu.make_async_copy(dst_hbm.at[wid, 0], dst_v, sem1).wait()
        else:
            pltpu.sync_copy(gidx_hbm.at[wid, h], gidx_v)
            pltpu.sync_copy(dst_hbm.at[wid, h], dst_v)
        for b in range(2):
            pltpu.async_copy(y2_hbm.at[gidx_v.at[b]], bufs[b][0], bufs[b][1])

        def pair(i, carry):
            for b in range(2):
                j = 2 * i + b
                buf, sem = bufs[b]
                pltpu.make_async_copy(y2_hbm.at[gidx_v.at[j]], buf, sem).wait()
                pltpu.sync_copy(buf, h_sh.at[dst_v.at[j]], add=True)

                @pl.when(j + 2 < CPP)
                def _():
                    pltpu.async_copy(y2_hbm.at[gidx_v.at[j + 2]], buf, sem)

            return carry

        lax.fori_loop(0, CPP // 2, pair, 0)
    plsc.subcore_barrier()

    @pl.when(sid < NS - 1)
    def _():
        pltpu.sync_copy(h_sh.at[pl.ds(rbase, ZB)], out_hbm.at[cid].at[pl.ds(rbase, ZB)])

    @pl.when(sid == NS - 1)
    def _():
        pltpu.sync_copy(h_sh.at[pl.ds(rbase, ZL)], out_hbm.at[cid].at[pl.ds(rbase, ZL)])


@functools.cache
def _sc_scatter_kernel():
    return pl.kernel(
        _sc_body,
        out_type=jax.ShapeDtypeStruct((NC, N, D), jnp.float32),
        mesh=plsc.VectorSubcoreMesh(
            core_axis_name="c", subcore_axis_name="s", num_cores=NC, num_subcores=NS
        ),
        scratch_types=[
            pltpu.VMEM((CPP, K), jnp.int32),
            pltpu.VMEM((CPP, K), jnp.int32),
            pltpu.VMEM((K, D), jnp.float32),
            pltpu.VMEM((K, D), jnp.float32),
            pltpu.VMEM_SHARED((NDUMMY, D), jnp.float32),
            pltpu.SemaphoreType.DMA,
            pltpu.SemaphoreType.DMA,
        ],
    )


FBN = 2000


def _final_body(p_ref, hl_ref, out_ref):
    out_ref[...] = p_ref[0] + p_ref[1] + hl_ref[...]


def _final(partial, h_loop):
    return pl.pallas_call(
        _final_body,
        grid=(N // FBN,),
        in_specs=[
            pl.BlockSpec((NC, FBN, D), lambda i: (0, i, 0)),
            pl.BlockSpec((FBN, D), lambda i: (i, 0)),
        ],
        out_specs=pl.BlockSpec((FBN, D), lambda i: (i, 0)),
        out_shape=jax.ShapeDtypeStruct((N, D), jnp.float32),
    )(partial, h_loop)


def kernel(x, edge_index, etypes, weight, w_comp, loop_weight, h_bias):
    y2, h_loop = _prep(x, w_comp, weight, loop_weight, h_bias.reshape(1, D))
    gidx, dstp = _gidx(
        edge_index.reshape(2, E // D, D),
        etypes.reshape(E // D, D),
    )
    partial = _sc_scatter_kernel()(
        y2.reshape(N * R, D),
        gidx,
        dstp,
        jnp.zeros((ZB, D), jnp.float32),
    )
    return _final(partial, h_loop)
